# Initial kernel scaffold; baseline (speedup 1.0000x reference)
import jax
import jax.numpy as jnp
from jax.experimental import pallas as pl


def kernel(logits):
    raise NotImplementedError("write your pallas kernel here")


# speedup vs baseline: 555.7473x; 555.7473x over previous
import functools

import numpy as np

import jax
import jax.numpy as jnp
from jax import lax
from jax.experimental import pallas as pl
from jax.experimental.pallas import tpu as pltpu
from jax.experimental.pallas import tpu_sc as plsc

B = 128
V = 100000
K = 50
TOPP = 0.9
L = 16
NW = 32
RPW = B // NW
GROUP = 10
GELEM = GROUP * L
NGRP = V // GELEM
SBGRP = 16
NCHUNK = 4
CHUNK = 24960
GPC = CHUNK // GELEM
NSB = GPC // SBGRP
SBREM = GPC - NSB * SBGRP
NGRPA = NCHUNK * GPC
G624OFF = NGRPA * GELEM
GMAXN = 640
NGV = NGRPA // L
PRUNE_AT = 192
CAP = PRUNE_AT + GELEM
NCV = CAP // L
C2 = 80
NC2 = C2 // L
C3 = 80
NC3 = 4
BIGI = np.int32(1 << 29)
NEG = np.float32(-np.inf)
NEGBIG = np.float32(-3.0e38)
KSENT = np.int32(-2**31)

_U1 = np.uint32(1)
_U31 = np.uint32(31)


def _iota():
    return lax.iota(jnp.int32, L)


def _f32key(v):
    bu = plsc.bitcast(v, jnp.uint32)
    sign = bu >> _U31
    flip = jnp.where(sign == _U1, jnp.uint32(0x7FFFFFFF), jnp.uint32(0))
    return plsc.bitcast(bu ^ flip, jnp.int32)


def _keyu(kv):
    return plsc.bitcast(kv, jnp.uint32) ^ jnp.uint32(0x80000000)


def _key2f32(kvec):
    top = kvec >> _U31
    bu = jnp.where(top == _U1, kvec ^ jnp.uint32(0x80000000),
                   kvec ^ jnp.uint32(0xFFFFFFFF))
    return plsc.bitcast(bu, jnp.float32)


def _pc(m):
    return plsc.all_reduce_population_count(m)


def _sampler_body(logits_hbm, rr_hbm, out_hbm,
                  rb0, rb1, rb2, rb3, g624, gmax, wlist, cand_v, cand_i,
                  keyb, c2v, c2i, c2k, c3v, c3i, nucb, rrb, outb, sem):
    wid = lax.axis_index("s") * 2 + lax.axis_index("c")
    iota = _iota()
    bufs = (rb0, rb1, rb2, rb3)

    pltpu.sync_copy(rr_hbm.at[wid], rrb)
    rrvec = rrb[pl.ds(0, L)]

    def issue_row(row):
        for c in range(NCHUNK):
            pltpu.async_copy(
                logits_hbm.at[row, pl.ds(c * CHUNK, CHUNK)],
                bufs[c], sem)
        pltpu.async_copy(
            logits_hbm.at[row, pl.ds(G624OFF, GELEM)], g624, sem)

    def wait_chunk(row, c):
        pltpu.make_async_copy(
            logits_hbm.at[row, pl.ds(c * CHUNK, CHUNK)],
            bufs[c], sem).wait()

    def wait_g624(row):
        pltpu.make_async_copy(
            logits_hbm.at[row, pl.ds(G624OFF, GELEM)], g624, sem).wait()

    def bisect_kth(ref, nvreg, k_target, nbits):
        def bit_body(_, carry):
            cur, bitv = carry
            cand_t = cur | bitv
            acc = jnp.zeros((L,), jnp.int32)
            for i in range(nvreg):
                kk = _keyu(_f32key(ref[pl.ds(i * L, L)]))
                acc = acc + _pc(kk >= cand_t)
            cur = jnp.where(acc >= k_target, cand_t, cur)
            return cur, bitv >> _U1
        cur, _ = lax.fori_loop(0, nbits, bit_body,
                               (jnp.zeros((L,), jnp.uint32),
                                jnp.full((L,), np.uint32(0x80000000),
                                         jnp.uint32)))
        return cur

    def keys_from_cands(ncand):
        for i in range(NCV):
            vv = cand_v[pl.ds(i * L, L)]
            valid = (iota + (i * L)) < ncand
            keyb[pl.ds(i * L, L)] = jnp.where(valid, _f32key(vv), KSENT)

    def kbisect(ncand, k_target, nbits):
        def bit_body(_, carry):
            cur, bitv = carry
            cand_t = cur | bitv
            acc = jnp.zeros((L,), jnp.int32)
            for i in range(NCV):
                kk = _keyu(keyb[pl.ds(i * L, L)])
                acc = acc + _pc(kk >= cand_t)
            cur = jnp.where(acc >= k_target, cand_t, cur)
            return cur, bitv >> _U1
        cur, _ = lax.fori_loop(0, nbits, bit_body,
                               (jnp.zeros((L,), jnp.uint32),
                                jnp.full((L,), np.uint32(0x80000000),
                                         jnp.uint32)))
        return cur

    def prune(ncand):
        keys_from_cands(ncand)
        vcur = kbisect(ncand, K, 18)
        nc = jnp.zeros((L,), jnp.int32)
        for i in range(NCV):
            kk = _keyu(keyb[pl.ds(i * L, L)])
            m = kk >= vcur
            vv = cand_v[pl.ds(i * L, L)]
            ii = cand_i[pl.ds(i * L, L)]
            pos = nc + plsc.cumsum(jnp.where(m, 1, 0)) - 1
            plsc.store_scatter(cand_v, [pos], vv, mask=m)
            plsc.store_scatter(cand_i, [pos], ii, mask=m)
            nc = nc + _pc(m)
        return nc[0], _key2f32(vcur)[0]

    def group_max(cref, lbase):
        a = cref[pl.ds(lbase, L)]
        b = cref[pl.ds(lbase + 5 * L, L)]
        for u in range(1, 5):
            a = jnp.maximum(a, cref[pl.ds(lbase + u * L, L)])
            b = jnp.maximum(b, cref[pl.ds(lbase + (5 + u) * L, L)])
        return jnp.max(jnp.maximum(a, b), axis=0)

    def pass1_chunk(c):
        cref = bufs[c]
        def sb_body(s, _):
            gvec = jnp.full((L,), NEG, jnp.float32)
            for j in range(SBGRP):
                gm = group_max(cref, (s * SBGRP + j) * GELEM)
                gvec = jnp.where(iota == j, gm, gvec)
            gmax[pl.ds(c * GPC + s * SBGRP, L)] = gvec
            return 0
        lax.fori_loop(0, NSB, sb_body, 0)
        gvec = jnp.full((L,), NEG, jnp.float32)
        for j in range(SBREM):
            gm = group_max(cref, (NSB * SBGRP + j) * GELEM)
            gvec = jnp.where(iota == j, gm, gvec)
        gmax[pl.ds(c * GPC + NSB * SBGRP, L)] = gvec

    def row_body(r, outvec):
        row = wid * RPW + r
        for c in range(NCHUNK):
            wait_chunk(row, c)
            pass1_chunk(c)

        tg_key = bisect_kth(gmax, NGV, K, 18)
        t_gf = _key2f32(tg_key)[0]

        wc = jnp.zeros((L,), jnp.int32)
        wb = [jnp.zeros((L,), jnp.int32) for _ in range(NCHUNK - 1)]
        for i in range(NGV):
            gv = gmax[pl.ds(i * L, L)]
            q = gv >= t_gf
            ids = iota + (i * L)
            pos = wc + plsc.cumsum(jnp.where(q, 1, 0)) - 1
            plsc.store_scatter(wlist, [pos], ids, mask=q)
            wc = wc + _pc(q)
            for c in range(NCHUNK - 1):
                wb[c] = wb[c] + _pc(q & (ids < (c + 1) * GPC))

        def make_entry_body(c):
            buf = bufs[c]

            def entry_body(w, carry):
                ncand, t = carry
                gid = wlist[pl.ds(w, L)][0]
                ebase = (gid - c * GPC) * GELEM
                gbase = c * CHUNK + ebase
                ncv = jnp.full((L,), ncand, jnp.int32)
                for j in range(GROUP):
                    v = buf[pl.ds(ebase + j * L, L)]
                    m = v >= t
                    pos = ncv + plsc.cumsum(jnp.where(m, 1, 0)) - 1
                    plsc.store_scatter(cand_v, [pos], v, mask=m)
                    plsc.store_scatter(cand_i, [pos],
                                       iota + (gbase + j * L), mask=m)
                    ncv = ncv + _pc(m)
                ncand = ncv[0]
                return lax.cond(ncand > PRUNE_AT,
                                lambda cc: prune(cc[0]),
                                lambda cc: (cc[0], cc[1]),
                                (ncand, t))
            return entry_body

        carry2 = (jnp.int32(0), t_gf)
        bounds = [jnp.int32(0)] + [b[0] for b in wb] + [wc[0]]
        for c in range(NCHUNK):
            carry2 = lax.fori_loop(bounds[c], bounds[c + 1],
                                   make_entry_body(c), carry2)
        ncand, t_after = carry2

        wait_g624(row)
        ncv = jnp.full((L,), ncand, jnp.int32)
        for j in range(GROUP):
            v = g624[pl.ds(j * L, L)]
            m = v >= t_after
            pos = ncv + plsc.cumsum(jnp.where(m, 1, 0)) - 1
            plsc.store_scatter(cand_v, [pos], v, mask=m)
            plsc.store_scatter(cand_i, [pos],
                               iota + (G624OFF + j * L), mask=m)
            ncv = ncv + _pc(m)
        ncand = ncv[0]

        @pl.when(r + 1 < RPW)
        def _():
            issue_row(row + 1)

        keys_from_cands(ncand)
        vcur = kbisect(ncand, K, 32)
        for i in range(NC2):
            c2k[pl.ds(i * L, L)] = jnp.full((L,), KSENT, jnp.int32)
            c2i[pl.ds(i * L, L)] = jnp.full((L,), BIGI, jnp.int32)
        nc2 = jnp.zeros((L,), jnp.int32)
        for i in range(NCV):
            kraw = keyb[pl.ds(i * L, L)]
            kk = _keyu(kraw)
            m = kk >= vcur
            vv = cand_v[pl.ds(i * L, L)]
            ii = cand_i[pl.ds(i * L, L)]
            pos = nc2 + plsc.cumsum(jnp.where(m, 1, 0)) - 1
            m2 = m & (pos < C2)
            plsc.store_scatter(c2v, [pos], vv, mask=m2)
            plsc.store_scatter(c2i, [pos], ii, mask=m2)
            plsc.store_scatter(c2k, [pos], kraw, mask=m2)
            nc2 = nc2 + _pc(m)

        cnt_gt = jnp.zeros((L,), jnp.int32)
        for i in range(NC2):
            cnt_gt = cnt_gt + _pc(_keyu(c2k[pl.ds(i * L, L)]) > vcur)
        need = K - cnt_gt

        def tie_body(_, carry):
            cur, bitv = carry
            cand = cur | bitv
            acc = jnp.zeros((L,), jnp.int32)
            for i in range(NC2):
                tie = _keyu(c2k[pl.ds(i * L, L)]) == vcur
                acc = acc + _pc(tie & (c2i[pl.ds(i * L, L)] < cand))
            cur = jnp.where(acc < need, cand, cur)
            return cur, bitv >> 1
        tie_x, _ = lax.fori_loop(0, 17, tie_body,
                                 (jnp.zeros((L,), jnp.int32),
                                  jnp.full((L,), np.int32(1 << 16),
                                           jnp.int32)))

        for i in range(C3 // L):
            c3v[pl.ds(i * L, L)] = jnp.full((L,), NEG, jnp.float32)
            c3i[pl.ds(i * L, L)] = jnp.full((L,), BIGI, jnp.int32)
        nc3 = jnp.zeros((L,), jnp.int32)
        for i in range(NC2):
            kk = _keyu(c2k[pl.ds(i * L, L)])
            ii = c2i[pl.ds(i * L, L)]
            m = (kk > vcur) | ((kk == vcur) & (ii <= tie_x))
            vv = c2v[pl.ds(i * L, L)]
            pos = nc3 + plsc.cumsum(jnp.where(m, 1, 0)) - 1
            plsc.store_scatter(c3v, [pos], vv, mask=m)
            plsc.store_scatter(c3i, [pos], ii, mask=m)
            nc3 = nc3 + _pc(m)

        vs = [c3v[pl.ds(i * L, L)] for i in range(NC3)]
        ids = [c3i[pl.ds(i * L, L)] for i in range(NC3)]
        slots = [iota + (i * L) for i in range(NC3)]
        valid = [s < K for s in slots]

        mvec = jnp.where(valid[0], vs[0], NEGBIG)
        for i in range(1, NC3):
            mvec = jnp.maximum(mvec, jnp.where(valid[i], vs[i], NEGBIG))
        mrow = jnp.max(mvec, axis=0)

        es = [jnp.where(valid[i], jnp.exp(vs[i] - mrow), 0.0)
              for i in range(NC3)]
        zacc = es[0]
        for i in range(1, NC3):
            zacc = zacc + es[i]
        z1 = jnp.sum(zacc, axis=0)
        ps = [e / z1 for e in es]

        def nuc_body(j, nuc):
            vj = c3v[pl.ds(j, L)][0]
            ij = c3i[pl.ds(j, L)][0]
            sacc = jnp.zeros((L,), jnp.float32)
            for i in range(NC3):
                before = (vs[i] > vj) | ((vs[i] == vj) & (ids[i] < ij))
                sacc = sacc + jnp.where(before, ps[i], 0.0)
            keep = jnp.sum(sacc, axis=0) <= np.float32(TOPP)
            return tuple(nuc[i] | jnp.where((slots[i] == j) & keep, 1, 0)
                         for i in range(NC3))

        nuc = lax.fori_loop(0, K, nuc_body,
                            tuple(jnp.zeros((L,), jnp.int32)
                                  for _ in range(NC3)))

        z2acc = jnp.where(nuc[0] == 1, es[0], 0.0)
        for i in range(1, NC3):
            z2acc = z2acc + jnp.where(nuc[i] == 1, es[i], 0.0)
        z2 = jnp.sum(z2acc, axis=0)
        qs = [jnp.where(nuc[i] == 1, es[i] / z2, 0.0) for i in range(NC3)]
        for i in range(NC3):
            nucb[pl.ds(i * L, L)] = nuc[i]
        nucb[pl.ds(NC3 * L, L)] = jnp.zeros((L,), jnp.int32)

        rr_r = jnp.sum(jnp.where(iota == r, rrvec, 0.0), axis=0)

        def samp_body(j, ans):
            ij = c3i[pl.ds(j, L)][0]
            nj = nucb[pl.ds(j, L)][0]
            tacc = jnp.zeros((L,), jnp.float32)
            for i in range(NC3):
                tacc = tacc + jnp.where(ids[i] <= ij, qs[i], 0.0)
            tj = jnp.sum(tacc, axis=0)
            hit = (nj == 1) & (tj > rr_r)
            return jnp.minimum(ans, jnp.where(hit, ij, BIGI))

        ans = lax.fori_loop(0, K, samp_body, jnp.int32(V))

        return jnp.where(iota == r, ans, outvec)

    issue_row(wid * RPW)
    outvec = lax.fori_loop(0, RPW, row_body, jnp.zeros((L,), jnp.int32))
    outb[pl.ds(0, L)] = outvec
    pltpu.sync_copy(outb, out_hbm.at[wid])


@functools.cache
def _build_sampler():
    mesh = plsc.VectorSubcoreMesh(core_axis_name="c", subcore_axis_name="s")
    return functools.partial(
        pl.kernel,
        out_type=jax.ShapeDtypeStruct((NW, L), jnp.int32),
        mesh=mesh,
        compiler_params=pltpu.CompilerParams(needs_layout_passes=False),
        scratch_types=[
            pltpu.VMEM((CHUNK,), jnp.float32),
            pltpu.VMEM((CHUNK,), jnp.float32),
            pltpu.VMEM((CHUNK,), jnp.float32),
            pltpu.VMEM((CHUNK,), jnp.float32),
            pltpu.VMEM((GELEM,), jnp.float32),
            pltpu.VMEM((GMAXN,), jnp.float32),
            pltpu.VMEM((GMAXN,), jnp.int32),
            pltpu.VMEM((CAP,), jnp.float32),
            pltpu.VMEM((CAP,), jnp.int32),
            pltpu.VMEM((CAP,), jnp.int32),
            pltpu.VMEM((C2,), jnp.float32),
            pltpu.VMEM((C2,), jnp.int32),
            pltpu.VMEM((C2,), jnp.int32),
            pltpu.VMEM((C3,), jnp.float32),
            pltpu.VMEM((C3,), jnp.int32),
            pltpu.VMEM((C3,), jnp.int32),
            pltpu.VMEM((L,), jnp.float32),
            pltpu.VMEM((L,), jnp.int32),
            pltpu.SemaphoreType.DMA,
        ],
    )(_sampler_body)


def kernel(logits):
    rr = jax.random.uniform(jax.random.key(1), (B, 1), dtype=jnp.float32)
    rr_pad = jnp.zeros((NW, L), jnp.float32).at[:, :RPW].set(
        rr.reshape(NW, RPW))
    out = _build_sampler()(logits, rr_pad)
    return out[:, :RPW].reshape(B, 1)


# speedup vs baseline: 641.2232x; 1.1538x over previous
import functools

import numpy as np

import jax
import jax.numpy as jnp
from jax import lax
from jax.experimental import pallas as pl
from jax.experimental.pallas import tpu as pltpu
from jax.experimental.pallas import tpu_sc as plsc

B = 128
V = 100000
K = 50
TOPP = 0.9
L = 16
NW = 32
RPW = B // NW
GROUP = 10
GELEM = GROUP * L
NGRP = V // GELEM
SBGRP = 16
NCHUNK = 4
CHUNK = 24960
GPC = CHUNK // GELEM
NSB = GPC // SBGRP
SBREM = GPC - NSB * SBGRP
NGRPA = NCHUNK * GPC
G624OFF = NGRPA * GELEM
GMAXN = 640
NGV = NGRPA // L
PRUNE_AT = 192
CAP = PRUNE_AT + GELEM
NCV = CAP // L
C2 = 80
NC2 = C2 // L
C3 = 80
NC3 = 4
BIGI = np.int32(1 << 29)
NEG = np.float32(-np.inf)
NEGBIG = np.float32(-3.0e38)
KSENT = np.int32(-2**31)

_U1 = np.uint32(1)
_U31 = np.uint32(31)


def _iota():
    return lax.iota(jnp.int32, L)


def _f32key(v):
    bu = plsc.bitcast(v, jnp.uint32)
    sign = bu >> _U31
    flip = jnp.where(sign == _U1, jnp.uint32(0x7FFFFFFF), jnp.uint32(0))
    return plsc.bitcast(bu ^ flip, jnp.int32)


def _keyu(kv):
    return plsc.bitcast(kv, jnp.uint32) ^ jnp.uint32(0x80000000)


def _key2f32(kvec):
    top = kvec >> _U31
    bu = jnp.where(top == _U1, kvec ^ jnp.uint32(0x80000000),
                   kvec ^ jnp.uint32(0xFFFFFFFF))
    return plsc.bitcast(bu, jnp.float32)


def _pc(m):
    return plsc.all_reduce_population_count(m)


def _sampler_body(logits_hbm, rr_hbm, out_hbm,
                  rb0, rb1, rb2, rb3, g624, gmax, wlist, cand_v, cand_i,
                  keyb, c2v, c2i, c2k, c3v, c3i, nucb, rrb, outb, sem):
    wid = lax.axis_index("s") * 2 + lax.axis_index("c")
    iota = _iota()
    bufs = (rb0, rb1, rb2, rb3)

    pltpu.sync_copy(rr_hbm.at[wid], rrb)
    rrvec = rrb[pl.ds(0, L)]

    def issue_row(row):
        for c in range(NCHUNK):
            pltpu.async_copy(
                logits_hbm.at[row, pl.ds(c * CHUNK, CHUNK)],
                bufs[c], sem)
        pltpu.async_copy(
            logits_hbm.at[row, pl.ds(G624OFF, GELEM)], g624, sem)

    def wait_chunk(row, c):
        pltpu.make_async_copy(
            logits_hbm.at[row, pl.ds(c * CHUNK, CHUNK)],
            bufs[c], sem).wait()

    def wait_g624(row):
        pltpu.make_async_copy(
            logits_hbm.at[row, pl.ds(G624OFF, GELEM)], g624, sem).wait()

    def bisect_kth(ref, nvreg, k_target, nbits):
        def bit_body(_, carry):
            cur, bitv = carry
            cand_t = cur | bitv
            acc = jnp.zeros((L,), jnp.int32)
            for i in range(nvreg):
                kk = _keyu(_f32key(ref[pl.ds(i * L, L)]))
                acc = acc + _pc(kk >= cand_t)
            cur = jnp.where(acc >= k_target, cand_t, cur)
            return cur, bitv >> _U1
        cur, _ = lax.fori_loop(0, nbits, bit_body,
                               (jnp.zeros((L,), jnp.uint32),
                                jnp.full((L,), np.uint32(0x80000000),
                                         jnp.uint32)))
        return cur

    def keys_from_cands(ncand):
        for i in range(NCV):
            vv = cand_v[pl.ds(i * L, L)]
            valid = (iota + (i * L)) < ncand
            keyb[pl.ds(i * L, L)] = jnp.where(valid, _f32key(vv), KSENT)

    def kbisect(ncand, k_target, nbits):
        def bit_body(_, carry):
            cur, bitv = carry
            cand_t = cur | bitv
            acc = jnp.zeros((L,), jnp.int32)
            for i in range(NCV):
                kk = _keyu(keyb[pl.ds(i * L, L)])
                acc = acc + _pc(kk >= cand_t)
            cur = jnp.where(acc >= k_target, cand_t, cur)
            return cur, bitv >> _U1
        cur, _ = lax.fori_loop(0, nbits, bit_body,
                               (jnp.zeros((L,), jnp.uint32),
                                jnp.full((L,), np.uint32(0x80000000),
                                         jnp.uint32)))
        return cur

    def prune(ncand):
        keys_from_cands(ncand)
        vcur = kbisect(ncand, K, 18)
        nc = jnp.zeros((L,), jnp.int32)
        for i in range(NCV):
            kk = _keyu(keyb[pl.ds(i * L, L)])
            m = kk >= vcur
            vv = cand_v[pl.ds(i * L, L)]
            ii = cand_i[pl.ds(i * L, L)]
            pos = nc + plsc.cumsum(jnp.where(m, 1, 0)) - 1
            plsc.store_scatter(cand_v, [pos], vv, mask=m)
            plsc.store_scatter(cand_i, [pos], ii, mask=m)
            nc = nc + _pc(m)
        return nc[0], _key2f32(vcur)[0]

    def group_max(cref, lbase):
        a = cref[pl.ds(lbase, L)]
        b = cref[pl.ds(lbase + 5 * L, L)]
        for u in range(1, 5):
            a = jnp.maximum(a, cref[pl.ds(lbase + u * L, L)])
            b = jnp.maximum(b, cref[pl.ds(lbase + (5 + u) * L, L)])
        return jnp.max(jnp.maximum(a, b), axis=0)

    def pass1_chunk(c):
        cref = bufs[c]
        def sb_body(s, _):
            gvec = jnp.full((L,), NEG, jnp.float32)
            for j in range(SBGRP):
                gm = group_max(cref, (s * SBGRP + j) * GELEM)
                gvec = jnp.where(iota == j, gm, gvec)
            gmax[pl.ds(c * GPC + s * SBGRP, L)] = gvec
            return 0
        lax.fori_loop(0, NSB, sb_body, 0)
        gvec = jnp.full((L,), NEG, jnp.float32)
        for j in range(SBREM):
            gm = group_max(cref, (NSB * SBGRP + j) * GELEM)
            gvec = jnp.where(iota == j, gm, gvec)
        gmax[pl.ds(c * GPC + NSB * SBGRP, L)] = gvec

    def row_body(r, outvec):
        row = wid * RPW + r
        for c in range(NCHUNK):
            wait_chunk(row, c)
            pass1_chunk(c)

        tg_key = bisect_kth(gmax, NGV, K, 18)
        t_gf = _key2f32(tg_key)[0]

        wc = jnp.zeros((L,), jnp.int32)
        wb = [jnp.zeros((L,), jnp.int32) for _ in range(NCHUNK - 1)]
        for i in range(NGV):
            gv = gmax[pl.ds(i * L, L)]
            q = gv >= t_gf
            ids = iota + (i * L)
            pos = wc + plsc.cumsum(jnp.where(q, 1, 0)) - 1
            plsc.store_scatter(wlist, [pos], ids, mask=q)
            wc = wc + _pc(q)
            for c in range(NCHUNK - 1):
                wb[c] = wb[c] + _pc(q & (ids < (c + 1) * GPC))

        def make_entry_body(c):
            buf = bufs[c]

            def entry_body(w, carry):
                ncand, t = carry
                gid = wlist[pl.ds(w, L)][0]
                ebase = (gid - c * GPC) * GELEM
                gbase = c * CHUNK + ebase
                vvs = [buf[pl.ds(ebase + j * L, L)] for j in range(GROUP)]
                ms = [v >= t for v in vvs]
                pcs = [_pc(m)[0] for m in ms]
                offs = [ncand]
                for j in range(GROUP - 1):
                    offs.append(offs[-1] + pcs[j])
                for j in range(GROUP):
                    pos = offs[j] + plsc.cumsum(jnp.where(ms[j], 1, 0)) - 1
                    plsc.store_scatter(cand_v, [pos], vvs[j], mask=ms[j])
                    plsc.store_scatter(cand_i, [pos],
                                       iota + (gbase + j * L), mask=ms[j])
                ncand = offs[-1] + pcs[-1]
                return lax.cond(ncand > PRUNE_AT,
                                lambda cc: prune(cc[0]),
                                lambda cc: (cc[0], cc[1]),
                                (ncand, t))
            return entry_body

        carry2 = (jnp.int32(0), t_gf)
        bounds = [jnp.int32(0)] + [b[0] for b in wb] + [wc[0]]
        for c in range(NCHUNK):
            carry2 = lax.fori_loop(bounds[c], bounds[c + 1],
                                   make_entry_body(c), carry2)
        ncand, t_after = carry2

        wait_g624(row)
        vvs = [g624[pl.ds(j * L, L)] for j in range(GROUP)]
        ms = [v >= t_after for v in vvs]
        pcs = [_pc(m)[0] for m in ms]
        offs = [ncand]
        for j in range(GROUP - 1):
            offs.append(offs[-1] + pcs[j])
        for j in range(GROUP):
            pos = offs[j] + plsc.cumsum(jnp.where(ms[j], 1, 0)) - 1
            plsc.store_scatter(cand_v, [pos], vvs[j], mask=ms[j])
            plsc.store_scatter(cand_i, [pos],
                               iota + (G624OFF + j * L), mask=ms[j])
        ncand = offs[-1] + pcs[-1]

        @pl.when(r + 1 < RPW)
        def _():
            issue_row(row + 1)

        keys_from_cands(ncand)
        vcur = kbisect(ncand, K, 32)
        for i in range(NC2):
            c2k[pl.ds(i * L, L)] = jnp.full((L,), KSENT, jnp.int32)
            c2i[pl.ds(i * L, L)] = jnp.full((L,), BIGI, jnp.int32)
        nc2 = jnp.zeros((L,), jnp.int32)
        for i in range(NCV):
            kraw = keyb[pl.ds(i * L, L)]
            kk = _keyu(kraw)
            m = kk >= vcur
            vv = cand_v[pl.ds(i * L, L)]
            ii = cand_i[pl.ds(i * L, L)]
            pos = nc2 + plsc.cumsum(jnp.where(m, 1, 0)) - 1
            m2 = m & (pos < C2)
            plsc.store_scatter(c2v, [pos], vv, mask=m2)
            plsc.store_scatter(c2i, [pos], ii, mask=m2)
            plsc.store_scatter(c2k, [pos], kraw, mask=m2)
            nc2 = nc2 + _pc(m)

        cnt_gt = jnp.zeros((L,), jnp.int32)
        for i in range(NC2):
            cnt_gt = cnt_gt + _pc(_keyu(c2k[pl.ds(i * L, L)]) > vcur)
        need = K - cnt_gt

        def tie_body(_, carry):
            cur, bitv = carry
            cand = cur | bitv
            acc = jnp.zeros((L,), jnp.int32)
            for i in range(NC2):
                tie = _keyu(c2k[pl.ds(i * L, L)]) == vcur
                acc = acc + _pc(tie & (c2i[pl.ds(i * L, L)] < cand))
            cur = jnp.where(acc < need, cand, cur)
            return cur, bitv >> 1
        tie_x, _ = lax.fori_loop(0, 17, tie_body,
                                 (jnp.zeros((L,), jnp.int32),
                                  jnp.full((L,), np.int32(1 << 16),
                                           jnp.int32)))

        for i in range(C3 // L):
            c3v[pl.ds(i * L, L)] = jnp.full((L,), NEG, jnp.float32)
            c3i[pl.ds(i * L, L)] = jnp.full((L,), BIGI, jnp.int32)
        nc3 = jnp.zeros((L,), jnp.int32)
        for i in range(NC2):
            kk = _keyu(c2k[pl.ds(i * L, L)])
            ii = c2i[pl.ds(i * L, L)]
            m = (kk > vcur) | ((kk == vcur) & (ii <= tie_x))
            vv = c2v[pl.ds(i * L, L)]
            pos = nc3 + plsc.cumsum(jnp.where(m, 1, 0)) - 1
            plsc.store_scatter(c3v, [pos], vv, mask=m)
            plsc.store_scatter(c3i, [pos], ii, mask=m)
            nc3 = nc3 + _pc(m)

        vs = [c3v[pl.ds(i * L, L)] for i in range(NC3)]
        ids = [c3i[pl.ds(i * L, L)] for i in range(NC3)]
        slots = [iota + (i * L) for i in range(NC3)]
        valid = [s < K for s in slots]

        mvec = jnp.where(valid[0], vs[0], NEGBIG)
        for i in range(1, NC3):
            mvec = jnp.maximum(mvec, jnp.where(valid[i], vs[i], NEGBIG))
        mrow = jnp.max(mvec, axis=0)

        es = [jnp.where(valid[i], jnp.exp(vs[i] - mrow), 0.0)
              for i in range(NC3)]
        zacc = es[0]
        for i in range(1, NC3):
            zacc = zacc + es[i]
        z1 = jnp.sum(zacc, axis=0)
        ps = [e / z1 for e in es]

        def nuc_body(j, nuc):
            vj = c3v[pl.ds(j, L)][0]
            ij = c3i[pl.ds(j, L)][0]
            sacc = jnp.zeros((L,), jnp.float32)
            for i in range(NC3):
                before = (vs[i] > vj) | ((vs[i] == vj) & (ids[i] < ij))
                sacc = sacc + jnp.where(before, ps[i], 0.0)
            keep = jnp.sum(sacc, axis=0) <= np.float32(TOPP)
            return tuple(nuc[i] | jnp.where((slots[i] == j) & keep, 1, 0)
                         for i in range(NC3))

        nuc = lax.fori_loop(0, K, nuc_body,
                            tuple(jnp.zeros((L,), jnp.int32)
                                  for _ in range(NC3)))

        z2acc = jnp.where(nuc[0] == 1, es[0], 0.0)
        for i in range(1, NC3):
            z2acc = z2acc + jnp.where(nuc[i] == 1, es[i], 0.0)
        z2 = jnp.sum(z2acc, axis=0)
        qs = [jnp.where(nuc[i] == 1, es[i] / z2, 0.0) for i in range(NC3)]
        for i in range(NC3):
            nucb[pl.ds(i * L, L)] = nuc[i]
        nucb[pl.ds(NC3 * L, L)] = jnp.zeros((L,), jnp.int32)

        rr_r = jnp.sum(jnp.where(iota == r, rrvec, 0.0), axis=0)

        def samp_body(j, ans):
            ij = c3i[pl.ds(j, L)][0]
            nj = nucb[pl.ds(j, L)][0]
            tacc = jnp.zeros((L,), jnp.float32)
            for i in range(NC3):
                tacc = tacc + jnp.where(ids[i] <= ij, qs[i], 0.0)
            tj = jnp.sum(tacc, axis=0)
            hit = (nj == 1) & (tj > rr_r)
            return jnp.minimum(ans, jnp.where(hit, ij, BIGI))

        ans = lax.fori_loop(0, K, samp_body, jnp.int32(V))

        return jnp.where(iota == r, ans, outvec)

    issue_row(wid * RPW)
    outvec = lax.fori_loop(0, RPW, row_body, jnp.zeros((L,), jnp.int32))
    outb[pl.ds(0, L)] = outvec
    pltpu.sync_copy(outb, out_hbm.at[wid])


@functools.cache
def _build_sampler():
    mesh = plsc.VectorSubcoreMesh(core_axis_name="c", subcore_axis_name="s")
    return functools.partial(
        pl.kernel,
        out_type=jax.ShapeDtypeStruct((NW, L), jnp.int32),
        mesh=mesh,
        compiler_params=pltpu.CompilerParams(needs_layout_passes=False),
        scratch_types=[
            pltpu.VMEM((CHUNK,), jnp.float32),
            pltpu.VMEM((CHUNK,), jnp.float32),
            pltpu.VMEM((CHUNK,), jnp.float32),
            pltpu.VMEM((CHUNK,), jnp.float32),
            pltpu.VMEM((GELEM,), jnp.float32),
            pltpu.VMEM((GMAXN,), jnp.float32),
            pltpu.VMEM((GMAXN,), jnp.int32),
            pltpu.VMEM((CAP,), jnp.float32),
            pltpu.VMEM((CAP,), jnp.int32),
            pltpu.VMEM((CAP,), jnp.int32),
            pltpu.VMEM((C2,), jnp.float32),
            pltpu.VMEM((C2,), jnp.int32),
            pltpu.VMEM((C2,), jnp.int32),
            pltpu.VMEM((C3,), jnp.float32),
            pltpu.VMEM((C3,), jnp.int32),
            pltpu.VMEM((C3,), jnp.int32),
            pltpu.VMEM((L,), jnp.float32),
            pltpu.VMEM((L,), jnp.int32),
            pltpu.SemaphoreType.DMA,
        ],
    )(_sampler_body)


def _rotl(x, r):
    return ((x << np.uint32(r)) | (x >> np.uint32(32 - r))).astype(np.uint32)


def _threefry2x32_np(k1, k2, x1, x2):
    rotations = ((13, 15, 26, 6), (17, 29, 16, 24))
    ks = [np.uint32(k1), np.uint32(k2),
          np.uint32(k1) ^ np.uint32(k2) ^ np.uint32(0x1BD11BDA)]
    x1 = (x1 + ks[0]).astype(np.uint32)
    x2 = (x2 + ks[1]).astype(np.uint32)
    for i in range(5):
        for r in rotations[i % 2]:
            x1 = (x1 + x2).astype(np.uint32)
            x2 = _rotl(x2, r)
            x2 = x2 ^ x1
        x1 = (x1 + ks[(i + 1) % 3]).astype(np.uint32)
        x2 = (x2 + ks[(i + 2) % 3] + np.uint32(i + 1)).astype(np.uint32)
    return x1, x2


def _rr_pad_const():
    o1, o2 = _threefry2x32_np(0, 1, np.zeros(B, np.uint32),
                              np.arange(B, dtype=np.uint32))
    bits = o1 ^ o2
    u = ((bits >> np.uint32(9)) | np.uint32(0x3F800000)).view(np.float32)
    u = np.maximum(np.float32(0.0), u - np.float32(1.0))
    pad = np.zeros((NW, L), np.float32)
    pad[:, :RPW] = u.reshape(NW, RPW)
    return pad


_RR_PAD = _rr_pad_const()


def kernel(logits):
    out = _build_sampler()(logits, jnp.asarray(_RR_PAD))
    return out[:, :RPW].reshape(B, 1)


# speedup vs baseline: 660.1054x; 1.0294x over previous
import functools

import numpy as np

import jax
import jax.numpy as jnp
from jax import lax
from jax.experimental import pallas as pl
from jax.experimental.pallas import tpu as pltpu
from jax.experimental.pallas import tpu_sc as plsc

B = 128
V = 100000
K = 50
TOPP = 0.9
L = 16
NW = 32
RPW = B // NW
GROUP = 10
GELEM = GROUP * L
NGRP = V // GELEM
SBGRP = 16
NCHUNK = 4
CHUNK = 24960
GPC = CHUNK // GELEM
CSIZE3 = CHUNK + GELEM
SBB = 12
NBAT = GPC // SBB
GMAXN = 640
NGV = 40
PRUNE_AT = 192
CAP = PRUNE_AT + GELEM
NCV = CAP // L
C2 = 80
NC2 = C2 // L
C3 = 80
NC3 = 4
BIGI = np.int32(1 << 29)
NEG = np.float32(-np.inf)
NEGBIG = np.float32(-3.0e38)
KSENT = np.int32(-2**31)

_U1 = np.uint32(1)
_U31 = np.uint32(31)


def _iota():
    return lax.iota(jnp.int32, L)


def _f32key(v):
    bu = plsc.bitcast(v, jnp.uint32)
    sign = bu >> _U31
    flip = jnp.where(sign == _U1, jnp.uint32(0x7FFFFFFF), jnp.uint32(0))
    return plsc.bitcast(bu ^ flip, jnp.int32)


def _keyu(kv):
    return plsc.bitcast(kv, jnp.uint32) ^ jnp.uint32(0x80000000)


def _key2f32(kvec):
    top = kvec >> _U31
    bu = jnp.where(top == _U1, kvec ^ jnp.uint32(0x80000000),
                   kvec ^ jnp.uint32(0xFFFFFFFF))
    return plsc.bitcast(bu, jnp.float32)


def _pc(m):
    return plsc.all_reduce_population_count(m)


def _sampler_body(logits_hbm, rr_hbm, out_hbm,
                  rb0, rb1, rb2, rb3, gmax, wlist, cand_v, cand_i,
                  keyb, c2v, c2i, c2k, c3v, c3i, nucb, rrb, outb, sem):
    wid = lax.axis_index("s") * 2 + lax.axis_index("c")
    iota = _iota()
    bufs = (rb0, rb1, rb2, rb3)

    pltpu.sync_copy(rr_hbm.at[wid], rrb)
    rrvec = rrb[pl.ds(0, L)]

    def chunk_size(c):
        return CSIZE3 if c == NCHUNK - 1 else CHUNK

    def issue_row(row):
        for c in range(NCHUNK):
            pltpu.async_copy(
                logits_hbm.at[row, pl.ds(c * CHUNK, chunk_size(c))],
                bufs[c], sem)

    def wait_chunk(row, c):
        pltpu.make_async_copy(
            logits_hbm.at[row, pl.ds(c * CHUNK, chunk_size(c))],
            bufs[c], sem).wait()

    def bisect_kth(ref, nvreg, k_target, nbits):
        def bit_body(_, carry):
            cur, bitv = carry
            cand_t = cur | bitv
            acc = jnp.zeros((L,), jnp.int32)
            for i in range(nvreg):
                kk = _keyu(_f32key(ref[pl.ds(i * L, L)]))
                if i == nvreg - 1:
                    kk = jnp.where(iota < 1, kk, jnp.uint32(0))
                acc = acc + _pc(kk >= cand_t)
            cur = jnp.where(acc >= k_target, cand_t, cur)
            return cur, bitv >> _U1
        cur, _ = lax.fori_loop(0, nbits, bit_body,
                               (jnp.zeros((L,), jnp.uint32),
                                jnp.full((L,), np.uint32(0x80000000),
                                         jnp.uint32)))
        return cur

    def keys_from_cands(ncand):
        for i in range(NCV):
            vv = cand_v[pl.ds(i * L, L)]
            valid = (iota + (i * L)) < ncand
            keyb[pl.ds(i * L, L)] = jnp.where(valid, _f32key(vv), KSENT)

    def kbisect(ncand, k_target, nbits):
        def bit_body(_, carry):
            cur, bitv = carry
            cand_t = cur | bitv
            acc = jnp.zeros((L,), jnp.int32)
            for i in range(NCV):
                kk = _keyu(keyb[pl.ds(i * L, L)])
                acc = acc + _pc(kk >= cand_t)
            cur = jnp.where(acc >= k_target, cand_t, cur)
            return cur, bitv >> _U1
        cur, _ = lax.fori_loop(0, nbits, bit_body,
                               (jnp.zeros((L,), jnp.uint32),
                                jnp.full((L,), np.uint32(0x80000000),
                                         jnp.uint32)))
        return cur

    def prune(ncand):
        keys_from_cands(ncand)
        vcur = kbisect(ncand, K, 18)
        nc = jnp.zeros((L,), jnp.int32)
        for i in range(NCV):
            kk = _keyu(keyb[pl.ds(i * L, L)])
            m = kk >= vcur
            vv = cand_v[pl.ds(i * L, L)]
            ii = cand_i[pl.ds(i * L, L)]
            pos = nc + plsc.cumsum(jnp.where(m, 1, 0)) - 1
            plsc.store_scatter(cand_v, [pos], vv, mask=m)
            plsc.store_scatter(cand_i, [pos], ii, mask=m)
            nc = nc + _pc(m)
        return nc[0], _key2f32(vcur)[0]

    def group_max(cref, lbase):
        a = cref[pl.ds(lbase, L)]
        b = cref[pl.ds(lbase + 5 * L, L)]
        for u in range(1, 5):
            a = jnp.maximum(a, cref[pl.ds(lbase + u * L, L)])
            b = jnp.maximum(b, cref[pl.ds(lbase + (5 + u) * L, L)])
        return jnp.max(jnp.maximum(a, b), axis=0)

    def pass1_chunk(c):
        cref = bufs[c]
        def sb_body(s, _):
            gvec = jnp.full((L,), NEG, jnp.float32)
            for j in range(SBB):
                gm = group_max(cref, (s * SBB + j) * GELEM)
                gvec = jnp.where(iota == j, gm, gvec)
            gmax[pl.ds(c * GPC + s * SBB, L)] = gvec
            return 0
        lax.fori_loop(0, NBAT, sb_body, 0)
        if c == NCHUNK - 1:
            gvec = jnp.full((L,), NEG, jnp.float32)
            gvec = jnp.where(iota < 1, group_max(cref, GPC * GELEM), gvec)
            gmax[pl.ds(NGRP - 1, L)] = gvec

    def row_body(r, outvec):
        row = wid * RPW + r
        for c in range(NCHUNK):
            wait_chunk(row, c)
            pass1_chunk(c)

        tg_key = bisect_kth(gmax, NGV, K, 18)
        t_gf = _key2f32(tg_key)[0]

        wc = jnp.zeros((L,), jnp.int32)
        for i in range(NGV):
            gv = gmax[pl.ds(i * L, L)]
            q = gv >= t_gf
            if i == NGV - 1:
                q = q & (iota < 1)
            pos = wc + plsc.cumsum(jnp.where(q, 1, 0)) - 1
            plsc.store_scatter(wlist, [pos], iota + (i * L), mask=q)
            wc = wc + _pc(q)

        def load_group(cix, lbase):
            def ld(c):
                return lambda: [bufs[c][pl.ds(lbase + j * L, L)]
                                for j in range(GROUP)]
            return lax.cond(
                cix < 2,
                lambda: lax.cond(cix < 1, ld(0), ld(1)),
                lambda: lax.cond(cix < 3, ld(2), ld(3)))

        def entry_body(w, carry):
            ncand, t = carry
            gid = wlist[pl.ds(w, L)][0]
            cix = jnp.minimum(gid // GPC, NCHUNK - 1)
            lbase = (gid - cix * GPC) * GELEM
            gbase = cix * CHUNK + lbase
            vvs = load_group(cix, lbase)
            ms = [v >= t for v in vvs]
            pcs = [_pc(m)[0] for m in ms]
            offs = [ncand]
            for j in range(GROUP - 1):
                offs.append(offs[-1] + pcs[j])
            for j in range(GROUP):
                pos = offs[j] + plsc.cumsum(jnp.where(ms[j], 1, 0)) - 1
                plsc.store_scatter(cand_v, [pos], vvs[j], mask=ms[j])
                plsc.store_scatter(cand_i, [pos],
                                   iota + (gbase + j * L), mask=ms[j])
            ncand = offs[-1] + pcs[-1]
            return lax.cond(ncand > PRUNE_AT,
                            lambda cc: prune(cc[0]),
                            lambda cc: (cc[0], cc[1]),
                            (ncand, t))

        ncand, _t = lax.fori_loop(0, wc[0], entry_body,
                                  (jnp.int32(0), t_gf))

        @pl.when(r + 1 < RPW)
        def _():
            issue_row(row + 1)

        keys_from_cands(ncand)
        vcur = kbisect(ncand, K, 32)
        for i in range(NC2):
            c2k[pl.ds(i * L, L)] = jnp.full((L,), KSENT, jnp.int32)
            c2i[pl.ds(i * L, L)] = jnp.full((L,), BIGI, jnp.int32)
        nc2 = jnp.zeros((L,), jnp.int32)
        for i in range(NCV):
            kraw = keyb[pl.ds(i * L, L)]
            kk = _keyu(kraw)
            m = kk >= vcur
            vv = cand_v[pl.ds(i * L, L)]
            ii = cand_i[pl.ds(i * L, L)]
            pos = nc2 + plsc.cumsum(jnp.where(m, 1, 0)) - 1
            m2 = m & (pos < C2)
            plsc.store_scatter(c2v, [pos], vv, mask=m2)
            plsc.store_scatter(c2i, [pos], ii, mask=m2)
            plsc.store_scatter(c2k, [pos], kraw, mask=m2)
            nc2 = nc2 + _pc(m)

        cnt_gt = jnp.zeros((L,), jnp.int32)
        for i in range(NC2):
            cnt_gt = cnt_gt + _pc(_keyu(c2k[pl.ds(i * L, L)]) > vcur)
        need = K - cnt_gt

        def tie_body(_, carry):
            cur, bitv = carry
            cand = cur | bitv
            acc = jnp.zeros((L,), jnp.int32)
            for i in range(NC2):
                tie = _keyu(c2k[pl.ds(i * L, L)]) == vcur
                acc = acc + _pc(tie & (c2i[pl.ds(i * L, L)] < cand))
            cur = jnp.where(acc < need, cand, cur)
            return cur, bitv >> 1
        tie_x, _ = lax.fori_loop(0, 17, tie_body,
                                 (jnp.zeros((L,), jnp.int32),
                                  jnp.full((L,), np.int32(1 << 16),
                                           jnp.int32)))

        for i in range(C3 // L):
            c3v[pl.ds(i * L, L)] = jnp.full((L,), NEG, jnp.float32)
            c3i[pl.ds(i * L, L)] = jnp.full((L,), BIGI, jnp.int32)
        nc3 = jnp.zeros((L,), jnp.int32)
        for i in range(NC2):
            kk = _keyu(c2k[pl.ds(i * L, L)])
            ii = c2i[pl.ds(i * L, L)]
            m = (kk > vcur) | ((kk == vcur) & (ii <= tie_x))
            vv = c2v[pl.ds(i * L, L)]
            pos = nc3 + plsc.cumsum(jnp.where(m, 1, 0)) - 1
            plsc.store_scatter(c3v, [pos], vv, mask=m)
            plsc.store_scatter(c3i, [pos], ii, mask=m)
            nc3 = nc3 + _pc(m)

        vs = [c3v[pl.ds(i * L, L)] for i in range(NC3)]
        ids = [c3i[pl.ds(i * L, L)] for i in range(NC3)]
        slots = [iota + (i * L) for i in range(NC3)]
        valid = [s < K for s in slots]

        mvec = jnp.where(valid[0], vs[0], NEGBIG)
        for i in range(1, NC3):
            mvec = jnp.maximum(mvec, jnp.where(valid[i], vs[i], NEGBIG))
        mrow = jnp.max(mvec, axis=0)

        es = [jnp.where(valid[i], jnp.exp(vs[i] - mrow), 0.0)
              for i in range(NC3)]
        zacc = es[0]
        for i in range(1, NC3):
            zacc = zacc + es[i]
        z1 = jnp.sum(zacc, axis=0)
        ps = [e / z1 for e in es]

        def nuc_body(j, nuc):
            vj = c3v[pl.ds(j, L)][0]
            ij = c3i[pl.ds(j, L)][0]
            sacc = jnp.zeros((L,), jnp.float32)
            for i in range(NC3):
                before = (vs[i] > vj) | ((vs[i] == vj) & (ids[i] < ij))
                sacc = sacc + jnp.where(before, ps[i], 0.0)
            keep = jnp.sum(sacc, axis=0) <= np.float32(TOPP)
            return tuple(nuc[i] | jnp.where((slots[i] == j) & keep, 1, 0)
                         for i in range(NC3))

        nuc = lax.fori_loop(0, K, nuc_body,
                            tuple(jnp.zeros((L,), jnp.int32)
                                  for _ in range(NC3)))

        z2acc = jnp.where(nuc[0] == 1, es[0], 0.0)
        for i in range(1, NC3):
            z2acc = z2acc + jnp.where(nuc[i] == 1, es[i], 0.0)
        z2 = jnp.sum(z2acc, axis=0)
        qs = [jnp.where(nuc[i] == 1, es[i] / z2, 0.0) for i in range(NC3)]
        for i in range(NC3):
            nucb[pl.ds(i * L, L)] = nuc[i]
        nucb[pl.ds(NC3 * L, L)] = jnp.zeros((L,), jnp.int32)

        rr_r = jnp.sum(jnp.where(iota == r, rrvec, 0.0), axis=0)

        def samp_body(j, ans):
            ij = c3i[pl.ds(j, L)][0]
            nj = nucb[pl.ds(j, L)][0]
            tacc = jnp.zeros((L,), jnp.float32)
            for i in range(NC3):
                tacc = tacc + jnp.where(ids[i] <= ij, qs[i], 0.0)
            tj = jnp.sum(tacc, axis=0)
            hit = (nj == 1) & (tj > rr_r)
            return jnp.minimum(ans, jnp.where(hit, ij, BIGI))

        ans = lax.fori_loop(0, K, samp_body, jnp.int32(V))

        return jnp.where(iota == r, ans, outvec)

    issue_row(wid * RPW)
    outvec = lax.fori_loop(0, RPW, row_body, jnp.zeros((L,), jnp.int32))
    outb[pl.ds(0, L)] = outvec
    pltpu.sync_copy(outb, out_hbm.at[wid])


@functools.cache
def _build_sampler():
    mesh = plsc.VectorSubcoreMesh(core_axis_name="c", subcore_axis_name="s")
    return functools.partial(
        pl.kernel,
        out_type=jax.ShapeDtypeStruct((NW, L), jnp.int32),
        mesh=mesh,
        compiler_params=pltpu.CompilerParams(needs_layout_passes=False),
        scratch_types=[
            pltpu.VMEM((CHUNK,), jnp.float32),
            pltpu.VMEM((CHUNK,), jnp.float32),
            pltpu.VMEM((CHUNK,), jnp.float32),
            pltpu.VMEM((CSIZE3,), jnp.float32),
            pltpu.VMEM((GMAXN,), jnp.float32),
            pltpu.VMEM((GMAXN,), jnp.int32),
            pltpu.VMEM((CAP,), jnp.float32),
            pltpu.VMEM((CAP,), jnp.int32),
            pltpu.VMEM((CAP,), jnp.int32),
            pltpu.VMEM((C2,), jnp.float32),
            pltpu.VMEM((C2,), jnp.int32),
            pltpu.VMEM((C2,), jnp.int32),
            pltpu.VMEM((C3,), jnp.float32),
            pltpu.VMEM((C3,), jnp.int32),
            pltpu.VMEM((C3,), jnp.int32),
            pltpu.VMEM((L,), jnp.float32),
            pltpu.VMEM((L,), jnp.int32),
            pltpu.SemaphoreType.DMA,
        ],
    )(_sampler_body)


def _rotl(x, r):
    return ((x << np.uint32(r)) | (x >> np.uint32(32 - r))).astype(np.uint32)


def _threefry2x32_np(k1, k2, x1, x2):
    rotations = ((13, 15, 26, 6), (17, 29, 16, 24))
    ks = [np.uint32(k1), np.uint32(k2),
          np.uint32(k1) ^ np.uint32(k2) ^ np.uint32(0x1BD11BDA)]
    x1 = (x1 + ks[0]).astype(np.uint32)
    x2 = (x2 + ks[1]).astype(np.uint32)
    for i in range(5):
        for r in rotations[i % 2]:
            x1 = (x1 + x2).astype(np.uint32)
            x2 = _rotl(x2, r)
            x2 = x2 ^ x1
        x1 = (x1 + ks[(i + 1) % 3]).astype(np.uint32)
        x2 = (x2 + ks[(i + 2) % 3] + np.uint32(i + 1)).astype(np.uint32)
    return x1, x2


def _rr_pad_const():
    o1, o2 = _threefry2x32_np(0, 1, np.zeros(B, np.uint32),
                              np.arange(B, dtype=np.uint32))
    bits = o1 ^ o2
    u = ((bits >> np.uint32(9)) | np.uint32(0x3F800000)).view(np.float32)
    u = np.maximum(np.float32(0.0), u - np.float32(1.0))
    pad = np.zeros((NW, L), np.float32)
    pad[:, :RPW] = u.reshape(NW, RPW)
    return pad


_RR_PAD = _rr_pad_const()


def kernel(logits):
    out = _build_sampler()(logits, jnp.asarray(_RR_PAD))
    return out[:, :RPW].reshape(B, 1)
